# Initial kernel scaffold; baseline (speedup 1.0000x reference)
import jax
import jax.numpy as jnp
from jax.experimental import pallas as pl


def kernel(x, edge_index, W1, b1, W2, b2, node_selection, mu, dyn_w):
    raise NotImplementedError("write your pallas kernel here")


# speedup vs baseline: 10.9536x; 10.9536x over previous
import functools

import jax
import jax.numpy as jnp
from jax import lax
from jax.experimental import pallas as pl
from jax.experimental.pallas import tpu as pltpu
from jax.experimental.pallas import tpu_sc as plsc

N = 10000
E = 320000
D = 128
H3 = 384
T_MAX = 5
ALPHA = 0.5

NC = 2
NS = 16
NW = NC * NS
B = 128
CHUNKS = 79
EPW = B * CHUNKS
E_PAD = EPW * NW
PAD = E_PAD - E
N_PAD = 10240
SPARE = N_PAD - N
NPT = N_PAD // NS

_mesh = plsc.VectorSubcoreMesh(core_axis_name="c", subcore_axis_name="s")


def _wid():
    return lax.axis_index("c") * NS + lax.axis_index("s")


def _zero_1d(z_v, accs):
    sid = lax.axis_index("s")
    for j in range(NPT // 16):
        z_v[pl.ds(j * 16, 16)] = jnp.zeros((16,), jnp.float32)
    for acc in accs:
        pltpu.sync_copy(z_v, acc.at[pl.ds(sid * NPT, NPT)])


def _degrs_body(row_s, dw0, dw1, dw2, dw3, dw4, cnt,
                a0, a1, a2, a3, a4, a5, idx_v, val_v, one_v, z_v):
    cid = lax.axis_index("c")
    sid = lax.axis_index("s")
    wid = _wid()
    accs = (a0, a1, a2, a3, a4, a5)
    dws = (dw0, dw1, dw2, dw3, dw4)
    _zero_1d(z_v, accs)
    for j in range(8):
        one_v[pl.ds(j * 16, 16)] = jnp.ones((16,), jnp.float32)
    plsc.subcore_barrier()

    def chunk(i, carry):
        base = wid * EPW + i * B
        pltpu.sync_copy(row_s.at[pl.ds(base, B)], idx_v)
        pltpu.sync_copy(one_v, a0.at[idx_v], add=True)
        for t in range(T_MAX):
            pltpu.sync_copy(dws[t].at[pl.ds(base, B)], val_v)
            for j in range(8):
                sl = pl.ds(j * 16, 16)
                val_v[sl] = jnp.maximum(val_v[sl], 0.0) + 1e-5
            pltpu.sync_copy(val_v, accs[t + 1].at[idx_v], add=True)
        return carry

    lax.fori_loop(0, CHUNKS, chunk, 0)
    plsc.subcore_barrier()

    @pl.when(sid == 0)
    def _():
        for t in range(6):
            pltpu.sync_copy(accs[t], cnt.at[cid, t])


def _degrs(row_s, dws):
    f = pl.kernel(
        _degrs_body,
        out_type=jax.ShapeDtypeStruct((NC, 6, N_PAD), jnp.float32),
        mesh=_mesh,
        scratch_types=[pltpu.VMEM_SHARED((N_PAD,), jnp.float32)] * 6 + [
            pltpu.VMEM((B,), jnp.int32),
            pltpu.VMEM((B,), jnp.float32),
            pltpu.VMEM((B,), jnp.float32),
            pltpu.VMEM((NPT,), jnp.float32),
        ],
    )
    return f(row_s, *dws)


def _prep_body(row_g, col_g, row_s, col_s, dis_s, oms, lapc, omr, ab,
               accA, accB, rg_v, cg_v, rs_v, cs_v, dr_v, dc_v, om_v, lc_v,
               ob_v, z_v, sem):
    cid = lax.axis_index("c")
    sid = lax.axis_index("s")
    wid = _wid()
    _zero_1d(z_v, (accA, accB))
    plsc.subcore_barrier()

    def chunk(i, carry):
        base = wid * EPW + i * B
        pltpu.sync_copy(row_g.at[pl.ds(base, B)], rg_v)
        pltpu.sync_copy(col_g.at[pl.ds(base, B)], cg_v)
        pltpu.sync_copy(row_s.at[pl.ds(base, B)], rs_v)
        pltpu.sync_copy(col_s.at[pl.ds(base, B)], cs_v)
        pltpu.async_copy(dis_s.at[rg_v], dr_v, sem).wait()
        pltpu.async_copy(dis_s.at[cg_v], dc_v, sem).wait()
        pltpu.async_copy(oms.at[rg_v], om_v, sem).wait()
        for j in range(8):
            sl = pl.ds(j * 16, 16)
            lc = dr_v[sl] * dc_v[sl]
            lc_v[sl] = lc
            ob_v[sl] = om_v[sl] * lc
        pltpu.sync_copy(lc_v, lapc.at[pl.ds(base, B)])
        pltpu.sync_copy(om_v, omr.at[pl.ds(base, B)])
        pltpu.sync_copy(lc_v, accA.at[rs_v], add=True)
        pltpu.sync_copy(ob_v, accB.at[cs_v], add=True)
        return carry

    lax.fori_loop(0, CHUNKS, chunk, 0)
    plsc.subcore_barrier()

    @pl.when(sid == 0)
    def _():
        pltpu.sync_copy(accA, ab.at[cid, 0])
        pltpu.sync_copy(accB, ab.at[cid, 1])


def _prep(row_g, col_g, row_s, col_s, dis_s, oms):
    f = pl.kernel(
        _prep_body,
        out_type=(
            jax.ShapeDtypeStruct((E_PAD,), jnp.float32),
            jax.ShapeDtypeStruct((E_PAD,), jnp.float32),
            jax.ShapeDtypeStruct((NC, 2, N_PAD), jnp.float32),
        ),
        mesh=_mesh,
        scratch_types=[
            pltpu.VMEM_SHARED((N_PAD,), jnp.float32),
            pltpu.VMEM_SHARED((N_PAD,), jnp.float32),
            pltpu.VMEM((B,), jnp.int32),
            pltpu.VMEM((B,), jnp.int32),
            pltpu.VMEM((B,), jnp.int32),
            pltpu.VMEM((B,), jnp.int32),
            pltpu.VMEM((B,), jnp.float32),
            pltpu.VMEM((B,), jnp.float32),
            pltpu.VMEM((B,), jnp.float32),
            pltpu.VMEM((B,), jnp.float32),
            pltpu.VMEM((B,), jnp.float32),
            pltpu.VMEM((NPT,), jnp.float32),
            pltpu.SemaphoreType.DMA,
        ],
    )
    return f(row_g, col_g, row_s, col_s, dis_s, oms)


def _msg_body(xt, col_g, row_g, row_s, col_s, dwt, lapc, omr, invrs,
              msgp, colwp, accM, accC, cg_v, rg_v, rs_v, cs_v, dw_v, lc_v,
              om_v, ir_v, c_v, cw_v, rows_v, zr_v, z_v, sem):
    cid = lax.axis_index("c")
    sid = lax.axis_index("s")
    wid = _wid()
    _zero_1d(z_v, (accC,))
    for i in range(16):
        for j in range(8):
            zr_v[i, pl.ds(j * 16, 16)] = jnp.zeros((16,), jnp.float32)
    for k in range(NPT // 16):
        pltpu.sync_copy(zr_v, accM.at[pl.ds(sid * NPT + k * 16, 16)])
    plsc.subcore_barrier()

    def chunk(i, carry):
        base = wid * EPW + i * B
        pltpu.sync_copy(col_g.at[pl.ds(base, B)], cg_v)
        pltpu.sync_copy(row_g.at[pl.ds(base, B)], rg_v)
        pltpu.sync_copy(row_s.at[pl.ds(base, B)], rs_v)
        pltpu.sync_copy(col_s.at[pl.ds(base, B)], cs_v)
        pltpu.sync_copy(dwt.at[pl.ds(base, B)], dw_v)
        pltpu.sync_copy(lapc.at[pl.ds(base, B)], lc_v)
        pltpu.sync_copy(omr.at[pl.ds(base, B)], om_v)
        pltpu.async_copy(invrs.at[rg_v], ir_v, sem).wait()
        pltpu.async_copy(xt.at[cg_v], rows_v, sem).wait()
        for j in range(8):
            sl = pl.ds(j * 16, 16)
            pos = jnp.maximum(dw_v[sl], 0.0) + 1e-5
            w = pos * ir_v[sl]
            c_v[sl] = w + lc_v[sl]
            cw_v[sl] = om_v[sl] * w
        for g in range(B // 16):
            cg = c_v[pl.ds(g * 16, 16)]
            for lane in range(16):
                e = g * 16 + lane
                cvec = jnp.full((16,), cg[lane], jnp.float32)
                for j in range(8):
                    sl = pl.ds(j * 16, 16)
                    rows_v[e, sl] = rows_v[e, sl] * cvec
        pltpu.sync_copy(rows_v, accM.at[rs_v], add=True)
        pltpu.sync_copy(cw_v, accC.at[cs_v], add=True)
        return carry

    lax.fori_loop(0, CHUNKS, chunk, 0)
    plsc.subcore_barrier()
    pltpu.sync_copy(accM.at[pl.ds(sid * NPT, NPT)],
                    msgp.at[cid, pl.ds(sid * NPT, NPT)])

    @pl.when(sid == 0)
    def _():
        pltpu.sync_copy(accC, colwp.at[cid])


def _msg(xt, col_g, row_g, row_s, col_s, dwt, lapc, omr, invrs):
    f = pl.kernel(
        _msg_body,
        out_type=(
            jax.ShapeDtypeStruct((NC, N_PAD, D), jnp.float32),
            jax.ShapeDtypeStruct((NC, N_PAD), jnp.float32),
        ),
        mesh=_mesh,
        scratch_types=[
            pltpu.VMEM_SHARED((N_PAD, D), jnp.float32),
            pltpu.VMEM_SHARED((N_PAD,), jnp.float32),
            pltpu.VMEM((B,), jnp.int32),
            pltpu.VMEM((B,), jnp.int32),
            pltpu.VMEM((B,), jnp.int32),
            pltpu.VMEM((B,), jnp.int32),
            pltpu.VMEM((B,), jnp.float32),
            pltpu.VMEM((B,), jnp.float32),
            pltpu.VMEM((B,), jnp.float32),
            pltpu.VMEM((B,), jnp.float32),
            pltpu.VMEM((B,), jnp.float32),
            pltpu.VMEM((B,), jnp.float32),
            pltpu.VMEM((B, D), jnp.float32),
            pltpu.VMEM((16, D), jnp.float32),
            pltpu.VMEM((NPT,), jnp.float32),
            pltpu.SemaphoreType.DMA,
        ],
    )
    return f(xt, col_g, row_g, row_s, col_s, dwt, lapc, omr, invrs)


def _mlp_body(x_ref, w1_ref, b1_ref, w2_ref, b2_ref, o_ref):
    h = jnp.dot(x_ref[...], w1_ref[...], preferred_element_type=jnp.float32)
    h = jnp.maximum(h + b1_ref[...], 0.0)
    o = jnp.dot(h, w2_ref[...], preferred_element_type=jnp.float32)
    o_ref[...] = o + b2_ref[...]


def _mlp(x, W1, b1, W2, b2):
    blk = 1000
    return pl.pallas_call(
        _mlp_body,
        grid=(N // blk,),
        in_specs=[
            pl.BlockSpec((blk, D), lambda i: (i, 0)),
            pl.BlockSpec((D, H3), lambda i: (0, 0)),
            pl.BlockSpec((1, H3), lambda i: (0, 0)),
            pl.BlockSpec((H3, D), lambda i: (0, 0)),
            pl.BlockSpec((1, D), lambda i: (0, 0)),
        ],
        out_specs=pl.BlockSpec((blk, D), lambda i: (i, 0)),
        out_shape=jax.ShapeDtypeStruct((N, D), jnp.float32),
    )(x, W1, b1.reshape(1, H3), W2, b2.reshape(1, D))


def _mid_body(cnt_ref, ns_ref, mu_ref, dis_ref, oms_ref, invrs_ref):
    mu = mu_ref[0, 0]
    deg = cnt_ref[0, 0, :] + cnt_ref[1, 0, :]
    safe = jnp.where(deg > 0, deg, 1.0)
    dis_ref[...] = (jnp.sqrt(mu) *
                    jnp.where(deg > 0, 1.0 / jnp.sqrt(safe), 0.0))[None, :]
    oms_ref[...] = (1.0 - 1.0 / (1.0 + jnp.exp(-ns_ref[...])))
    for t in range(T_MAX):
        rs = cnt_ref[0, t + 1, :] + cnt_ref[1, t + 1, :]
        invrs_ref[t, :] = 1.0 / (rs + 1e-8)


def _mid(cnt, ns_pad, mu2d):
    return pl.pallas_call(
        _mid_body,
        in_specs=[
            pl.BlockSpec((NC, 6, N_PAD), lambda: (0, 0, 0)),
            pl.BlockSpec((1, N_PAD), lambda: (0, 0)),
            pl.BlockSpec(memory_space=pltpu.SMEM),
        ],
        out_specs=(
            pl.BlockSpec((1, N_PAD), lambda: (0, 0)),
            pl.BlockSpec((1, N_PAD), lambda: (0, 0)),
            pl.BlockSpec((T_MAX, N_PAD), lambda: (0, 0)),
        ),
        out_shape=(
            jax.ShapeDtypeStruct((1, N_PAD), jnp.float32),
            jax.ShapeDtypeStruct((1, N_PAD), jnp.float32),
            jax.ShapeDtypeStruct((T_MAX, N_PAD), jnp.float32),
        ),
    )(cnt, ns_pad, mu2d)


def _upd_body(xt_ref, mp_ref, x0_ref, ns_ref, mu_ref, o_ref):
    mu = mu_ref[0, 0]
    xt = xt_ref[...]
    m = mp_ref[0] + mp_ref[1]
    s = 1.0 / (1.0 + jnp.exp(-ns_ref[...]))
    oms = 1.0 - s
    ni = m - mu * xt
    o_ref[...] = jnp.maximum(
        ALPHA * xt + (1.0 - ALPHA) * (s * x0_ref[...] + oms * ni), 0.0)


def _upd(xt, msgp, x0, ns2d, mu2d):
    blk = 1000
    return pl.pallas_call(
        _upd_body,
        grid=(N // blk,),
        in_specs=[
            pl.BlockSpec((blk, D), lambda i: (i, 0)),
            pl.BlockSpec((NC, blk, D), lambda i: (0, i, 0)),
            pl.BlockSpec((blk, D), lambda i: (i, 0)),
            pl.BlockSpec((blk, 1), lambda i: (i, 0)),
            pl.BlockSpec(memory_space=pltpu.SMEM),
        ],
        out_specs=pl.BlockSpec((blk, D), lambda i: (i, 0)),
        out_shape=jax.ShapeDtypeStruct((N, D), jnp.float32),
    )(xt, msgp, x0, ns2d, mu2d)


def _reg_body(cnt_ref, ab_ref, cw_ref, ns_ref, mu_ref, o_ref):
    mu = mu_ref[0, 0]
    oms = 1.0 - 1.0 / (1.0 + jnp.exp(-ns_ref[...]))
    a0 = (ab_ref[0, 0, :] + ab_ref[1, 0, :])[None, :]
    b0 = (ab_ref[0, 1, :] + ab_ref[1, 1, :])[None, :]
    nidx = lax.broadcasted_iota(jnp.int32, (1, N_PAD), 1)
    valid = nidx < N
    reg = jnp.float32(0.0)
    for t in range(T_MAX):
        rs = (cnt_ref[0, t + 1, :] + cnt_ref[1, t + 1, :])[None, :]
        ratio = rs / (rs + 1e-8)
        colw = (cw_ref[t, 0, :] + cw_ref[t, 1, :])[None, :]
        row_sums = jnp.where(valid, oms * (ratio + a0 + mu), 0.0)
        col_sums = jnp.where(valid, colw + b0 + mu * oms, 0.0)
        opn = jnp.sqrt(jnp.max(row_sums) * jnp.max(col_sums))
        reg = reg + jnp.maximum(opn - 1.0 + 1e-10, 0.0)
    o_ref[...] = jnp.full((1, 1), reg, jnp.float32)


def _reg(cnt, ab, colws, ns_pad, mu2d):
    return pl.pallas_call(
        _reg_body,
        in_specs=[
            pl.BlockSpec((NC, 6, N_PAD), lambda: (0, 0, 0)),
            pl.BlockSpec((NC, 2, N_PAD), lambda: (0, 0, 0)),
            pl.BlockSpec((T_MAX, NC, N_PAD), lambda: (0, 0, 0)),
            pl.BlockSpec((1, N_PAD), lambda: (0, 0)),
            pl.BlockSpec(memory_space=pltpu.SMEM),
        ],
        out_specs=pl.BlockSpec((1, 1), lambda: (0, 0)),
        out_shape=jax.ShapeDtypeStruct((1, 1), jnp.float32),
    )(cnt, ab, colws, ns_pad, mu2d)


def kernel(x, edge_index, W1, b1, W2, b2, node_selection, mu, dyn_w):
    row = edge_index[0]
    col = edge_index[1]
    pad_ids = jnp.arange(PAD, dtype=jnp.int32)
    row_g = jnp.concatenate([row, pad_ids % N])
    col_g = jnp.concatenate([col, (pad_ids * 131) % N])
    row_s = jnp.concatenate([row, N + pad_ids % SPARE])
    col_s = jnp.concatenate([col, N + pad_ids % SPARE])
    dw_pad = jnp.pad(dyn_w, ((0, 0), (0, PAD)))
    dws = [dw_pad[t] for t in range(T_MAX)]
    ns_pad = jnp.pad(node_selection, (0, N_PAD - N)).reshape(1, N_PAD)
    ns2d = node_selection.reshape(N, 1)
    mu2d = jnp.asarray(mu, jnp.float32).reshape(1, 1)

    x0 = _mlp(x, W1, b1, W2, b2)
    cnt = _degrs(row_s, dws)
    dis_s, oms, invrs = _mid(cnt, ns_pad, mu2d)
    dis_s1 = dis_s.reshape(N_PAD)
    oms1 = oms.reshape(N_PAD)
    lapc, omr, ab = _prep(row_g, col_g, row_s, col_s, dis_s1, oms1)

    xt = x0
    colw_list = []
    for t in range(T_MAX):
        msgp, colwp = _msg(xt, col_g, row_g, row_s, col_s, dws[t], lapc, omr,
                           invrs[t].reshape(N_PAD))
        xt = _upd(xt, msgp, x0, ns2d, mu2d)
        colw_list.append(colwp)

    colws = jnp.stack(colw_list)
    reg = _reg(cnt, ab, colws, ns_pad, mu2d)
    return xt, reg.reshape(())


# speedup vs baseline: 26.1383x; 2.3863x over previous
import functools

import jax
import jax.numpy as jnp
from jax import lax
from jax.experimental import pallas as pl
from jax.experimental.pallas import tpu as pltpu
from jax.experimental.pallas import tpu_sc as plsc

N = 10000
E = 320000
D = 128
H3 = 384
T_MAX = 5
ALPHA = 0.5

NC = 2
NS = 16
NW = NC * NS
EPW = 10240
E_PAD = EPW * NW
PAD = E_PAD - E
B = 128
CHUNKS = EPW // B
B2 = 128
CHUNKS2 = EPW // B2
N_PAD = 10240
SPARE = N_PAD - N
NPT = N_PAD // NS

_mesh = plsc.VectorSubcoreMesh(core_axis_name="c", subcore_axis_name="s")


def _wid():
    return lax.axis_index("c") * NS + lax.axis_index("s")


def _zero_1d(z_v, accs):
    sid = lax.axis_index("s")
    for j in range(NPT // 16):
        z_v[pl.ds(j * 16, 16)] = jnp.zeros((16,), jnp.float32)
    for acc in accs:
        pltpu.sync_copy(z_v, acc.at[pl.ds(sid * NPT, NPT)])


def _degrs_body(idxs, dw, cnt, a0, a1, a2, a3, a4, a5,
                idx_v, dw_v, one_v, v0, v1, v2, v3, v4, z_v, sem):
    cid = lax.axis_index("c")
    sid = lax.axis_index("s")
    wid = _wid()
    accs = (a0, a1, a2, a3, a4, a5)
    vals = (one_v, v0, v1, v2, v3, v4)
    _zero_1d(z_v, accs)
    for j in range(B2 // 16):
        one_v[pl.ds(j * 16, 16)] = jnp.ones((16,), jnp.float32)
    plsc.subcore_barrier()

    def chunk(i, carry):
        base = wid * EPW + i * B2
        pltpu.sync_copy(idxs.at[0, pl.ds(base, B2)], idx_v)
        pltpu.sync_copy(dw.at[:, pl.ds(base, B2)], dw_v)
        for t in range(T_MAX):
            for j in range(B2 // 16):
                sl = pl.ds(j * 16, 16)
                vals[t + 1][sl] = jnp.maximum(dw_v[t, sl], 0.0) + 1e-5
        descs = [pltpu.make_async_copy(vals[t], accs[t].at[idx_v], sem)
                 for t in range(6)]
        for d in descs:
            d.start(add=True)
        for d in descs:
            d.wait()
        return carry

    lax.fori_loop(0, CHUNKS2, chunk, 0)
    plsc.subcore_barrier()

    @pl.when(sid == 0)
    def _():
        for t in range(6):
            pltpu.sync_copy(accs[t], cnt.at[cid, t])


def _degrs(idxs, dw):
    f = pl.kernel(
        _degrs_body,
        out_type=jax.ShapeDtypeStruct((NC, 6, N_PAD), jnp.float32),
        mesh=_mesh,
        scratch_types=[pltpu.VMEM_SHARED((N_PAD,), jnp.float32)] * 6 + [
            pltpu.VMEM((B2,), jnp.int32),
            pltpu.VMEM((T_MAX, B2), jnp.float32),
        ] + [pltpu.VMEM((B2,), jnp.float32)] * 6 + [
            pltpu.VMEM((NPT,), jnp.float32),
            pltpu.SemaphoreType.DMA,
        ],
    )
    return f(idxs, dw)


def _prep_body(idxg, idxs, dis_s, oms, ed2, ab,
               accA, accB, ig_v, is_v, dr_v, dc_v, ed_v, ob_v, z_v,
               sg, ss):
    cid = lax.axis_index("c")
    sid = lax.axis_index("s")
    wid = _wid()
    _zero_1d(z_v, (accA, accB))
    plsc.subcore_barrier()

    def chunk(i, carry):
        base = wid * EPW + i * B2
        pltpu.sync_copy(idxg.at[:, pl.ds(base, B2)], ig_v)
        pltpu.sync_copy(idxs.at[:, pl.ds(base, B2)], is_v)
        gd = (pltpu.make_async_copy(dis_s.at[ig_v.at[1]], dr_v, sg),
              pltpu.make_async_copy(dis_s.at[ig_v.at[0]], dc_v, sg),
              pltpu.make_async_copy(oms.at[ig_v.at[1]], ed_v.at[1], sg))
        for d in gd:
            d.start()
        for d in gd:
            d.wait()
        for j in range(B2 // 16):
            sl = pl.ds(j * 16, 16)
            lc = dr_v[sl] * dc_v[sl]
            ed_v[0, sl] = lc
            ob_v[sl] = ed_v[1, sl] * lc
        pltpu.sync_copy(ed_v, ed2.at[:, pl.ds(base, B2)])
        sd = (pltpu.make_async_copy(ed_v.at[0], accA.at[is_v.at[0]], ss),
              pltpu.make_async_copy(ob_v, accB.at[is_v.at[1]], ss))
        for d in sd:
            d.start(add=True)
        for d in sd:
            d.wait()
        return carry

    lax.fori_loop(0, CHUNKS2, chunk, 0)
    plsc.subcore_barrier()

    @pl.when(sid == 0)
    def _():
        pltpu.sync_copy(accA, ab.at[cid, 0])
        pltpu.sync_copy(accB, ab.at[cid, 1])


def _prep(idxg, idxs, dis_s, oms):
    f = pl.kernel(
        _prep_body,
        out_type=(
            jax.ShapeDtypeStruct((2, E_PAD), jnp.float32),
            jax.ShapeDtypeStruct((NC, 2, N_PAD), jnp.float32),
        ),
        mesh=_mesh,
        scratch_types=[
            pltpu.VMEM_SHARED((N_PAD,), jnp.float32),
            pltpu.VMEM_SHARED((N_PAD,), jnp.float32),
            pltpu.VMEM((2, B2), jnp.int32),
            pltpu.VMEM((2, B2), jnp.int32),
            pltpu.VMEM((B2,), jnp.float32),
            pltpu.VMEM((B2,), jnp.float32),
            pltpu.VMEM((2, B2), jnp.float32),
            pltpu.VMEM((B2,), jnp.float32),
            pltpu.VMEM((NPT,), jnp.float32),
            pltpu.SemaphoreType.DMA,
            pltpu.SemaphoreType.DMA,
        ],
    )
    return f(idxg, idxs, dis_s, oms)


def _msg_body(xt, idxg, idxs, ed2, dwt, invrs, msgp, colwp,
              accM, accC,
              ig0, ig1, is0, is1, ed0, ed1, dv0, dv1, ir0, ir1,
              c0, c1, cw0, cw1, rw0, rw1, zr_v, z_v,
              slA0, slA1, slB0, slB1, sg0, sg1, sr0, sr1, sc0, sc1):
    cid = lax.axis_index("c")
    sid = lax.axis_index("s")
    wid = _wid()
    IG = (ig0, ig1); IS = (is0, is1); ED = (ed0, ed1); DV = (dv0, dv1)
    IR = (ir0, ir1); C = (c0, c1); CW = (cw0, cw1); RW = (rw0, rw1)
    SLA = (slA0, slA1); SLB = (slB0, slB1); SG = (sg0, sg1)
    SR = (sr0, sr1); SC = (sc0, sc1)

    _zero_1d(z_v, (accC,))
    for i in range(16):
        for j in range(D // 16):
            zr_v[i, pl.ds(j * 16, 16)] = jnp.zeros((16,), jnp.float32)
    for k in range(NPT // 16):
        pltpu.sync_copy(zr_v, accM.at[pl.ds(sid * NPT + k * 16, 16)])
    plsc.subcore_barrier()

    def cbase(k):
        return wid * EPW + lax.rem(k, CHUNKS) * B

    def linA(k, b):
        base = cbase(k)
        return (pltpu.make_async_copy(idxg.at[:, pl.ds(base, B)], IG[b], SLA[b]),
                pltpu.make_async_copy(ed2.at[:, pl.ds(base, B)], ED[b], SLA[b]),
                pltpu.make_async_copy(dwt.at[pl.ds(base, B)], DV[b], SLA[b]))

    def linB(k, b):
        base = cbase(k)
        return (pltpu.make_async_copy(idxs.at[:, pl.ds(base, B)], IS[b], SLB[b]),)

    def gath(b):
        return (pltpu.make_async_copy(invrs.at[IG[b].at[1]], IR[b], SG[b]),
                pltpu.make_async_copy(xt.at[IG[b].at[0]], RW[b], SG[b]))

    def scat(b):
        return (pltpu.make_async_copy(RW[b], accM.at[IS[b].at[0]], SR[b]),
                pltpu.make_async_copy(CW[b], accC.at[IS[b].at[1]], SC[b]))

    def compute(b):
        for j in range(B // 16):
            sl = pl.ds(j * 16, 16)
            pos = jnp.maximum(DV[b][sl], 0.0) + 1e-5
            w = pos * IR[b][sl]
            C[b][sl] = w + ED[b][0, sl]
            CW[b][sl] = ED[b][1, sl] * w
        for g2 in range(B // 16):
            cg = C[b][pl.ds(g2 * 16, 16)]
            for lane in range(16):
                e = g2 * 16 + lane
                cvec = jnp.full((16,), cg[lane], jnp.float32)
                for j in range(D // 16):
                    sl = pl.ds(j * 16, 16)
                    RW[b][e, sl] = RW[b][e, sl] * cvec

    for d in linA(0, 0):
        d.start()
    for d in linA(1, 1):
        d.start()
    for d in linB(0, 0):
        d.start()
    for d in linA(0, 0):
        d.wait()
    for d in gath(0):
        d.start()

    def step(g, i, b):
        ob = 1 - b
        for d in linA(i + 1, ob):
            d.wait()
        if b == 0:
            @pl.when(g > 0)
            def _():
                for d in scat(ob):
                    d.wait()
        else:
            for d in scat(ob):
                d.wait()
        for d in gath(ob):
            d.start()
        for d in linB(i + 1, ob):
            d.start()
        for d in gath(b):
            d.wait()
        compute(b)
        for d in linB(i, b):
            d.wait()
        for d in scat(b):
            d.start(add=True)
        for d in linA(i + 2, b):
            d.start()

    def macro(g, carry):
        step(g, 2 * g, 0)
        step(g, 2 * g + 1, 1)
        return carry

    lax.fori_loop(0, CHUNKS // 2, macro, 0)

    for d in scat(1):
        d.wait()
    for d in gath(0):
        d.wait()
    for d in linB(0, 0):
        d.wait()
    for d in linA(0, 1):
        d.wait()
    plsc.subcore_barrier()
    pltpu.sync_copy(accM.at[pl.ds(sid * NPT, NPT)],
                    msgp.at[cid, pl.ds(sid * NPT, NPT)])

    @pl.when(sid == 0)
    def _():
        pltpu.sync_copy(accC, colwp.at[cid])


def _msg(xt, idxg, idxs, ed2, dwt, invrs):
    f = pl.kernel(
        _msg_body,
        out_type=(
            jax.ShapeDtypeStruct((NC, N_PAD, D), jnp.float32),
            jax.ShapeDtypeStruct((NC, N_PAD), jnp.float32),
        ),
        mesh=_mesh,
        scratch_types=[
            pltpu.VMEM_SHARED((N_PAD, D), jnp.float32),
            pltpu.VMEM_SHARED((N_PAD,), jnp.float32),
            pltpu.VMEM((2, B), jnp.int32),
            pltpu.VMEM((2, B), jnp.int32),
            pltpu.VMEM((2, B), jnp.int32),
            pltpu.VMEM((2, B), jnp.int32),
            pltpu.VMEM((2, B), jnp.float32),
            pltpu.VMEM((2, B), jnp.float32),
            pltpu.VMEM((B,), jnp.float32),
            pltpu.VMEM((B,), jnp.float32),
            pltpu.VMEM((B,), jnp.float32),
            pltpu.VMEM((B,), jnp.float32),
            pltpu.VMEM((B,), jnp.float32),
            pltpu.VMEM((B,), jnp.float32),
            pltpu.VMEM((B,), jnp.float32),
            pltpu.VMEM((B,), jnp.float32),
            pltpu.VMEM((B, D), jnp.float32),
            pltpu.VMEM((B, D), jnp.float32),
            pltpu.VMEM((16, D), jnp.float32),
            pltpu.VMEM((NPT,), jnp.float32),
        ] + [pltpu.SemaphoreType.DMA] * 10,
    )
    return f(xt, idxg, idxs, ed2, dwt, invrs)


def _mlp_body(x_ref, w1_ref, b1_ref, w2_ref, b2_ref, o_ref):
    h = jnp.dot(x_ref[...], w1_ref[...], preferred_element_type=jnp.float32)
    h = jnp.maximum(h + b1_ref[...], 0.0)
    o = jnp.dot(h, w2_ref[...], preferred_element_type=jnp.float32)
    o_ref[...] = o + b2_ref[...]


def _mlp(x, W1, b1, W2, b2):
    blk = 1000
    return pl.pallas_call(
        _mlp_body,
        grid=(N // blk,),
        in_specs=[
            pl.BlockSpec((blk, D), lambda i: (i, 0)),
            pl.BlockSpec((D, H3), lambda i: (0, 0)),
            pl.BlockSpec((1, H3), lambda i: (0, 0)),
            pl.BlockSpec((H3, D), lambda i: (0, 0)),
            pl.BlockSpec((1, D), lambda i: (0, 0)),
        ],
        out_specs=pl.BlockSpec((blk, D), lambda i: (i, 0)),
        out_shape=jax.ShapeDtypeStruct((N, D), jnp.float32),
    )(x, W1, b1.reshape(1, H3), W2, b2.reshape(1, D))


def _mid_body(cnt_ref, ns_ref, mu_ref, dis_ref, oms_ref, invrs_ref):
    mu = mu_ref[0, 0]
    deg = cnt_ref[0, 0, :] + cnt_ref[1, 0, :]
    safe = jnp.where(deg > 0, deg, 1.0)
    dis_ref[...] = (jnp.sqrt(mu) *
                    jnp.where(deg > 0, 1.0 / jnp.sqrt(safe), 0.0))[None, :]
    oms_ref[...] = (1.0 - 1.0 / (1.0 + jnp.exp(-ns_ref[...])))
    for t in range(T_MAX):
        rs = cnt_ref[0, t + 1, :] + cnt_ref[1, t + 1, :]
        invrs_ref[t, :] = 1.0 / (rs + 1e-8)


def _mid(cnt, ns_pad, mu2d):
    return pl.pallas_call(
        _mid_body,
        in_specs=[
            pl.BlockSpec((NC, 6, N_PAD), lambda: (0, 0, 0)),
            pl.BlockSpec((1, N_PAD), lambda: (0, 0)),
            pl.BlockSpec(memory_space=pltpu.SMEM),
        ],
        out_specs=(
            pl.BlockSpec((1, N_PAD), lambda: (0, 0)),
            pl.BlockSpec((1, N_PAD), lambda: (0, 0)),
            pl.BlockSpec((T_MAX, N_PAD), lambda: (0, 0)),
        ),
        out_shape=(
            jax.ShapeDtypeStruct((1, N_PAD), jnp.float32),
            jax.ShapeDtypeStruct((1, N_PAD), jnp.float32),
            jax.ShapeDtypeStruct((T_MAX, N_PAD), jnp.float32),
        ),
    )(cnt, ns_pad, mu2d)


def _upd_body(xt_ref, mp_ref, x0_ref, ns_ref, mu_ref, o_ref):
    mu = mu_ref[0, 0]
    xt = xt_ref[...]
    m = mp_ref[0] + mp_ref[1]
    s = 1.0 / (1.0 + jnp.exp(-ns_ref[...]))
    oms = 1.0 - s
    ni = m - mu * xt
    o_ref[...] = jnp.maximum(
        ALPHA * xt + (1.0 - ALPHA) * (s * x0_ref[...] + oms * ni), 0.0)


def _upd(xt, msgp, x0, ns2d, mu2d):
    blk = 1000
    return pl.pallas_call(
        _upd_body,
        grid=(N // blk,),
        in_specs=[
            pl.BlockSpec((blk, D), lambda i: (i, 0)),
            pl.BlockSpec((NC, blk, D), lambda i: (0, i, 0)),
            pl.BlockSpec((blk, D), lambda i: (i, 0)),
            pl.BlockSpec((blk, 1), lambda i: (i, 0)),
            pl.BlockSpec(memory_space=pltpu.SMEM),
        ],
        out_specs=pl.BlockSpec((blk, D), lambda i: (i, 0)),
        out_shape=jax.ShapeDtypeStruct((N, D), jnp.float32),
    )(xt, msgp, x0, ns2d, mu2d)


def _reg_body(cnt_ref, ab_ref, cw_ref, ns_ref, mu_ref, o_ref):
    mu = mu_ref[0, 0]
    oms = 1.0 - 1.0 / (1.0 + jnp.exp(-ns_ref[...]))
    a0 = (ab_ref[0, 0, :] + ab_ref[1, 0, :])[None, :]
    b0 = (ab_ref[0, 1, :] + ab_ref[1, 1, :])[None, :]
    nidx = lax.broadcasted_iota(jnp.int32, (1, N_PAD), 1)
    valid = nidx < N
    reg = jnp.float32(0.0)
    for t in range(T_MAX):
        rs = (cnt_ref[0, t + 1, :] + cnt_ref[1, t + 1, :])[None, :]
        ratio = rs / (rs + 1e-8)
        colw = (cw_ref[t, 0, :] + cw_ref[t, 1, :])[None, :]
        row_sums = jnp.where(valid, oms * (ratio + a0 + mu), 0.0)
        col_sums = jnp.where(valid, colw + b0 + mu * oms, 0.0)
        opn = jnp.sqrt(jnp.max(row_sums) * jnp.max(col_sums))
        reg = reg + jnp.maximum(opn - 1.0 + 1e-10, 0.0)
    o_ref[...] = jnp.full((1, 1), reg, jnp.float32)


def _reg(cnt, ab, colws, ns_pad, mu2d):
    return pl.pallas_call(
        _reg_body,
        in_specs=[
            pl.BlockSpec((NC, 6, N_PAD), lambda: (0, 0, 0)),
            pl.BlockSpec((NC, 2, N_PAD), lambda: (0, 0, 0)),
            pl.BlockSpec((T_MAX, NC, N_PAD), lambda: (0, 0, 0)),
            pl.BlockSpec((1, N_PAD), lambda: (0, 0)),
            pl.BlockSpec(memory_space=pltpu.SMEM),
        ],
        out_specs=pl.BlockSpec((1, 1), lambda: (0, 0)),
        out_shape=jax.ShapeDtypeStruct((1, 1), jnp.float32),
    )(cnt, ab, colws, ns_pad, mu2d)


def kernel(x, edge_index, W1, b1, W2, b2, node_selection, mu, dyn_w):
    row = edge_index[0]
    col = edge_index[1]
    pad_ids = jnp.arange(PAD, dtype=jnp.int32)
    row_g = jnp.concatenate([row, pad_ids % N])
    col_g = jnp.concatenate([col, (pad_ids * 131) % N])
    row_s = jnp.concatenate([row, N + pad_ids % SPARE])
    col_s = jnp.concatenate([col, N + pad_ids % SPARE])
    idxg = jnp.stack([col_g, row_g])
    idxs = jnp.stack([row_s, col_s])
    dw_pad = jnp.pad(dyn_w, ((0, 0), (0, PAD)))
    ns_pad = jnp.pad(node_selection, (0, N_PAD - N)).reshape(1, N_PAD)
    ns2d = node_selection.reshape(N, 1)
    mu2d = jnp.asarray(mu, jnp.float32).reshape(1, 1)

    x0 = _mlp(x, W1, b1, W2, b2)
    cnt = _degrs(idxs, dw_pad)
    dis_s, oms, invrs = _mid(cnt, ns_pad, mu2d)
    ed2, ab = _prep(idxg, idxs, dis_s.reshape(N_PAD), oms.reshape(N_PAD))

    xt = x0
    colw_list = []
    for t in range(T_MAX):
        msgp, colwp = _msg(xt, idxg, idxs, ed2, dw_pad[t],
                           invrs[t].reshape(N_PAD))
        xt = _upd(xt, msgp, x0, ns2d, mu2d)
        colw_list.append(colwp)

    colws = jnp.stack(colw_list)
    reg = _reg(cnt, ab, colws, ns_pad, mu2d)
    return xt, reg.reshape(())
